# baseline (speedup 1.0000x reference)
import functools

import jax
import jax.numpy as jnp
from jax import lax
from jax.experimental import pallas as pl
from jax.experimental.pallas import tpu as pltpu
from jax.experimental.pallas import tpu_sc as plsc

N_NODE = 10000
N_RAD = 6
N_L = 10
N_C = 16
N_OUT = 4

N_ROWS = N_NODE * N_RAD
IN_W = N_L * N_C
OUT_W = N_OUT * N_C

NUM_WORKERS = 32
CHUNK = 120
NUM_CHUNKS = N_ROWS // CHUNK


def _row_poly(xv, ov, i):
    a = [xv[i, pl.ds(16 * l, 16)] for l in range(N_L)]
    s1 = a[1] * a[1]
    s2 = a[2] * a[2]
    s3 = a[3] * a[3]
    s4 = a[4] * a[4]
    s5 = a[5] * a[5]
    s6 = a[6] * a[6]
    s7 = a[7] * a[7]
    s8 = a[8] * a[8]
    s9 = a[9] * a[9]
    out1 = s1 + s2 + s3
    out2 = (s4 + s7 + s9) + 2.0 * (s5 + s6 + s8)
    u56 = s5 + s6
    u58 = s5 + s8
    u68 = s6 + s8
    cubes = a[4] * s4 + a[7] * s7 + a[9] * s9
    mixed = a[4] * u56 + a[7] * u58 + a[9] * u68
    triple = a[5] * a[6] * a[8]
    out3 = cubes + 3.0 * mixed + 6.0 * triple
    ov[i, pl.ds(0, 16)] = a[0]
    ov[i, pl.ds(16, 16)] = out1
    ov[i, pl.ds(32, 16)] = out2
    ov[i, pl.ds(48, 16)] = out3


def _sym_body(x_hbm, out_hbm, xv, ov):
    wid = lax.axis_index("s") * 2 + lax.axis_index("c")
    my_chunks = (NUM_CHUNKS - wid + NUM_WORKERS - 1) // NUM_WORKERS

    def chunk_body(ci, carry):
        row0 = (wid + ci * NUM_WORKERS) * CHUNK
        pltpu.sync_copy(x_hbm.at[pl.ds(row0, CHUNK)], xv)

        def row_body(i, c2):
            _row_poly(xv, ov, i)
            return c2

        lax.fori_loop(0, CHUNK, row_body, 0, unroll=4)
        pltpu.sync_copy(ov, out_hbm.at[pl.ds(row0, CHUNK)])
        return carry

    lax.fori_loop(0, my_chunks, chunk_body, 0)


_sym_call = functools.partial(
    pl.kernel,
    out_type=jax.ShapeDtypeStruct((N_ROWS, OUT_W), jnp.float32),
    mesh=plsc.VectorSubcoreMesh(core_axis_name="c", subcore_axis_name="s"),
    scratch_types=[
        pltpu.VMEM((CHUNK, IN_W), jnp.float32),
        pltpu.VMEM((CHUNK, OUT_W), jnp.float32),
    ],
)(_sym_body)


@jax.jit
def kernel(node_attr):
    x = node_attr.reshape(N_ROWS, IN_W)
    out = _sym_call(x)
    return out.reshape(N_NODE, N_RAD, N_OUT, N_C)


# speedup vs baseline: 1.0205x; 1.0205x over previous
import functools

import jax
import jax.numpy as jnp
from jax import lax
from jax.experimental import pallas as pl
from jax.experimental.pallas import tpu as pltpu
from jax.experimental.pallas import tpu_sc as plsc

N_NODE = 10000
N_RAD = 6
N_L = 10
N_C = 16
N_OUT = 4

N_ROWS = N_NODE * N_RAD
IN_W = N_L * N_C
OUT_W = N_OUT * N_C

NUM_WORKERS = 32
CHUNK = 120
NUM_CHUNKS = N_ROWS // CHUNK


def _row_poly(xv, ov, i):
    a = [xv[i, pl.ds(16 * l, 16)] for l in range(N_L)]
    s1 = a[1] * a[1]
    s2 = a[2] * a[2]
    s3 = a[3] * a[3]
    s4 = a[4] * a[4]
    s5 = a[5] * a[5]
    s6 = a[6] * a[6]
    s7 = a[7] * a[7]
    s8 = a[8] * a[8]
    s9 = a[9] * a[9]
    out1 = s1 + s2 + s3
    out2 = (s4 + s7 + s9) + 2.0 * (s5 + s6 + s8)
    u56 = s5 + s6
    u58 = s5 + s8
    u68 = s6 + s8
    cubes = a[4] * s4 + a[7] * s7 + a[9] * s9
    mixed = a[4] * u56 + a[7] * u58 + a[9] * u68
    triple = a[5] * a[6] * a[8]
    out3 = cubes + 3.0 * mixed + 6.0 * triple
    ov[i, pl.ds(0, 16)] = a[0]
    ov[i, pl.ds(16, 16)] = out1
    ov[i, pl.ds(32, 16)] = out2
    ov[i, pl.ds(48, 16)] = out3


def _sym_body(x_hbm, out_hbm, xv, ov):
    wid = lax.axis_index("s") * 2 + lax.axis_index("c")
    my_chunks = (NUM_CHUNKS - wid + NUM_WORKERS - 1) // NUM_WORKERS

    def chunk_body(ci, carry):
        row0 = (wid + ci * NUM_WORKERS) * CHUNK
        pltpu.sync_copy(x_hbm.at[pl.ds(row0, CHUNK)], xv)

        @plsc.parallel_loop(0, CHUNK, unroll=8)
        def row_body(i):
            _row_poly(xv, ov, i)
        pltpu.sync_copy(ov, out_hbm.at[pl.ds(row0, CHUNK)])
        return carry

    lax.fori_loop(0, my_chunks, chunk_body, 0)


_sym_call = functools.partial(
    pl.kernel,
    out_type=jax.ShapeDtypeStruct((N_ROWS, OUT_W), jnp.float32),
    mesh=plsc.VectorSubcoreMesh(core_axis_name="c", subcore_axis_name="s"),
    scratch_types=[
        pltpu.VMEM((CHUNK, IN_W), jnp.float32),
        pltpu.VMEM((CHUNK, OUT_W), jnp.float32),
    ],
)(_sym_body)


@jax.jit
def kernel(node_attr):
    x = node_attr.reshape(N_ROWS, IN_W)
    out = _sym_call(x)
    return out.reshape(N_NODE, N_RAD, N_OUT, N_C)


# speedup vs baseline: 25.4050x; 24.8957x over previous
import functools

import jax
import jax.numpy as jnp
from jax import lax
from jax.experimental import pallas as pl
from jax.experimental.pallas import tpu as pltpu

N_NODE = 10000
N_RAD = 6
N_L = 10
N_C = 16
N_OUT = 4

BN = 512
TC_GRID = (N_NODE + BN - 1) // BN


def _poly(a):
    s1 = a[1] * a[1]
    s2 = a[2] * a[2]
    s3 = a[3] * a[3]
    s4 = a[4] * a[4]
    s5 = a[5] * a[5]
    s6 = a[6] * a[6]
    s7 = a[7] * a[7]
    s8 = a[8] * a[8]
    s9 = a[9] * a[9]
    out1 = s1 + s2 + s3
    out2 = (s4 + s7 + s9) + 2.0 * (s5 + s6 + s8)
    cubes = a[4] * s4 + a[7] * s7 + a[9] * s9
    mixed = a[4] * (s5 + s6) + a[7] * (s5 + s8) + a[9] * (s6 + s8)
    triple = a[5] * a[6] * a[8]
    out3 = cubes + 3.0 * mixed + 6.0 * triple
    return a[0], out1, out2, out3


def _tc_body(x_ref, o_ref):
    a = [x_ref[:, l] for l in range(N_L)]
    o0, o1, o2, o3 = _poly(a)
    o_ref[:, 0] = o0
    o_ref[:, 1] = o1
    o_ref[:, 2] = o2
    o_ref[:, 3] = o3


_tc_call = pl.pallas_call(
    _tc_body,
    grid=(TC_GRID,),
    in_specs=[
        pl.BlockSpec((N_RAD, N_L, N_C, BN), lambda i: (0, 0, 0, i)),
    ],
    out_specs=pl.BlockSpec((N_RAD, N_OUT, N_C, BN), lambda i: (0, 0, 0, i)),
    out_shape=jax.ShapeDtypeStruct((N_RAD, N_OUT, N_C, N_NODE), jnp.float32),
)


@jax.jit
def kernel(node_attr):
    xt = jnp.transpose(node_attr, (1, 2, 3, 0))
    yt = _tc_call(xt)
    return jnp.transpose(yt, (3, 0, 1, 2))
